# Initial kernel scaffold; baseline (speedup 1.0000x reference)
import jax
import jax.numpy as jnp
from jax.experimental import pallas as pl


def kernel(x, Wr, br, W1, b1, W2, b2, W3, b3, SW1, sb1, SW2, sb2, SW3, sb3):
    raise NotImplementedError("write your pallas kernel here")


# speedup vs baseline: 3.4304x; 3.4304x over previous
import functools

import jax
import jax.numpy as jnp
from jax.experimental import pallas as pl
from jax.experimental.pallas import tpu as pltpu

B, S, D, H, E, K = 2, 2048, 1024, 2048, 8, 2
Z_COEF = 0.001
N = B * S
EE = E + 1

BM = 1024
BH = 512
NB_M = N // BM
NB_H = H // BH


def _gelu(x):
    return 0.5 * x * (1.0 + jax.lax.erf(x * 0.7071067811865476))


def _moe_body(x_ref, w1_ref, b1_ref, w2_ref, b2_ref, w3_ref, b3_ref,
              wts_ref, out_ref):
    e = pl.program_id(0)
    h = pl.program_id(1)
    m = pl.program_id(2)

    xb = x_ref[pl.ds(m * BM, BM), :]
    a = _gelu(jnp.dot(xb, w1_ref[0].T, preferred_element_type=jnp.float32)
              + b1_ref[0, 0][None, :])
    g = _gelu(jnp.dot(xb, w3_ref[0].T, preferred_element_type=jnp.float32)
              + b3_ref[0, 0][None, :])
    contrib = jnp.dot(a * g, w2_ref[0].T, preferred_element_type=jnp.float32)

    onehot = (jax.lax.broadcasted_iota(jnp.int32, (1, EE), 1) == e)
    wcol = jnp.sum(wts_ref[pl.ds(m * BM, BM), :] * onehot.astype(jnp.float32),
                   axis=1, keepdims=True)

    delta = contrib * wcol

    @pl.when(h == 0)
    def _addb2():
        out_ref[pl.ds(m * BM, BM), :] = jnp.where(
            e == 0, 0.0, out_ref[pl.ds(m * BM, BM), :]) + wcol * b2_ref[0, 0][None, :]

    out_ref[pl.ds(m * BM, BM), :] += delta


@functools.partial(jax.jit, static_argnames=())
def _moe_dense(xf, W1c, b1c, W2c, b2c, W3c, b3c, wts):
    return pl.pallas_call(
        _moe_body,
        grid=(EE, NB_H, NB_M),
        in_specs=[
            pl.BlockSpec((N, D), lambda e, h, m: (0, 0)),
            pl.BlockSpec((1, BH, D), lambda e, h, m: (e, h, 0)),
            pl.BlockSpec((1, 1, BH), lambda e, h, m: (e, 0, h)),
            pl.BlockSpec((1, D, BH), lambda e, h, m: (e, 0, h)),
            pl.BlockSpec((1, 1, D), lambda e, h, m: (e, 0, 0)),
            pl.BlockSpec((1, BH, D), lambda e, h, m: (e, h, 0)),
            pl.BlockSpec((1, 1, BH), lambda e, h, m: (e, 0, h)),
            pl.BlockSpec((N, EE), lambda e, h, m: (0, 0)),
        ],
        out_specs=pl.BlockSpec((N, D), lambda e, h, m: (0, 0)),
        out_shape=jax.ShapeDtypeStruct((N, D), jnp.float32),
        compiler_params=pltpu.CompilerParams(
            dimension_semantics=("arbitrary", "arbitrary", "arbitrary"),
        ),
    )(xf, W1c, b1c, W2c, b2c, W3c, b3c, wts)


def kernel(x, Wr, br, W1, b1, W2, b2, W3, b3, SW1, sb1, SW2, sb2, SW3, sb3):
    xf = x.reshape(N, D)

    logits = xf @ Wr.T + br
    probs = jax.nn.softmax(logits, axis=-1)
    topv, topi = jax.lax.top_k(probs, K)
    rw = topv / jnp.sum(topv, axis=-1, keepdims=True)
    oh = jax.nn.one_hot(topi, E, dtype=x.dtype)
    wts = jnp.sum(rw[..., None] * oh, axis=1)
    wts = jnp.concatenate([wts, jnp.ones((N, 1), jnp.float32)], axis=1)

    usage = jnp.mean(probs, axis=0)
    aux_loss = jnp.sum(usage * usage) * E * Z_COEF

    W1c = jnp.concatenate([W1, SW1[None]], axis=0)
    b1c = jnp.concatenate([b1, sb1[None]], axis=0)[:, None, :]
    W2c = jnp.concatenate([W2, SW2[None]], axis=0)
    b2c = jnp.concatenate([b2, sb2[None]], axis=0)[:, None, :]
    W3c = jnp.concatenate([W3, SW3[None]], axis=0)
    b3c = jnp.concatenate([b3, sb3[None]], axis=0)[:, None, :]

    out = _moe_dense(xf, W1c, b1c, W2c, b2c, W3c, b3c, wts)
    return (out.reshape(B, S, D), aux_loss)
